# Initial kernel scaffold; baseline (speedup 1.0000x reference)
import jax
import jax.numpy as jnp
from jax.experimental import pallas as pl


def kernel(x):
    raise NotImplementedError("write your pallas kernel here")


# speedup vs baseline: 1.8097x; 1.8097x over previous
import jax
import jax.numpy as jnp
from jax.experimental import pallas as pl
from jax.experimental.pallas import tpu as pltpu

_CHUNK = 128
_ROWS_PER_BLOCK = 1024


def _scan_block_kernel(x_ref, o_ref, carry_ref):
    c = pl.program_id(1)

    @pl.when(c == 0)
    def _():
        carry_ref[...] = jnp.zeros_like(carry_ref)

    x = x_ref[...]
    n = x.shape[1]
    row_i = jax.lax.broadcasted_iota(jnp.int32, (n, n), 0)
    col_i = jax.lax.broadcasted_iota(jnp.int32, (n, n), 1)
    l_strict = (row_i < col_i).astype(x.dtype)
    part = jnp.dot(x, l_strict, preferred_element_type=jnp.float32)
    carry = carry_ref[...]
    o_ref[...] = part + carry
    carry_ref[...] = carry + jnp.sum(x, axis=1, keepdims=True)


def _exclusive_scan(x, rows_per_block=_ROWS_PER_BLOCK, chunk=_CHUNK,
                    interpret=False):
    n_rows, n_cols = x.shape
    out_rows = n_rows - 1
    out_cols = n_cols + 1
    n_row_blocks = pl.cdiv(out_rows, rows_per_block)
    n_col_blocks = n_cols // chunk + 1
    last_in_block = n_cols // chunk - 1
    return pl.pallas_call(
        _scan_block_kernel,
        grid=(n_row_blocks, n_col_blocks),
        in_specs=[
            pl.BlockSpec(
                (rows_per_block, chunk),
                lambda r, c: (r, jnp.minimum(c, last_in_block)),
            )
        ],
        out_specs=pl.BlockSpec((rows_per_block, chunk), lambda r, c: (r, c)),
        out_shape=jax.ShapeDtypeStruct((out_rows, out_cols), x.dtype),
        scratch_shapes=[pltpu.VMEM((rows_per_block, 1), jnp.float32)],
        compiler_params=pltpu.CompilerParams(
            dimension_semantics=("parallel", "arbitrary")
        ),
        interpret=interpret,
    )(x)


def kernel(x):
    return _exclusive_scan(x)


# speedup vs baseline: 3.1047x; 1.7156x over previous
import jax
import jax.numpy as jnp
from jax.experimental import pallas as pl
from jax.experimental.pallas import tpu as pltpu

_CHUNK = 128
_ROWS_PER_BLOCK = 256


def _scan_block_kernel(x_ref, o_ref):
    n_cols = x_ref.shape[1]
    n_chunks = n_cols // _CHUNK
    row_i = jax.lax.broadcasted_iota(jnp.int32, (_CHUNK, _CHUNK), 0)
    col_i = jax.lax.broadcasted_iota(jnp.int32, (_CHUNK, _CHUNK), 1)
    l_strict = (row_i < col_i).astype(jnp.float32)
    carry = jnp.zeros((x_ref.shape[0], 1), dtype=jnp.float32)
    for c in range(n_chunks):
        xc = x_ref[:, c * _CHUNK:(c + 1) * _CHUNK]
        part = jnp.dot(xc, l_strict, preferred_element_type=jnp.float32)
        o_ref[:, c * _CHUNK:(c + 1) * _CHUNK] = part + carry
        carry = carry + (part + xc)[:, _CHUNK - 1:_CHUNK]
    o_ref[:, n_cols:n_cols + 1] = carry


def _exclusive_scan(x, rows_per_block=_ROWS_PER_BLOCK, interpret=False):
    n_rows, n_cols = x.shape
    out_rows = n_rows - 1
    out_cols = n_cols + 1
    n_row_blocks = pl.cdiv(out_rows, rows_per_block)
    return pl.pallas_call(
        _scan_block_kernel,
        grid=(n_row_blocks,),
        in_specs=[pl.BlockSpec((rows_per_block, n_cols), lambda r: (r, 0))],
        out_specs=pl.BlockSpec((rows_per_block, out_cols), lambda r: (r, 0)),
        out_shape=jax.ShapeDtypeStruct((out_rows, out_cols), x.dtype),
        compiler_params=pltpu.CompilerParams(
            dimension_semantics=("parallel",)
        ),
        interpret=interpret,
    )(x)


def kernel(x):
    return _exclusive_scan(x)
